# baseline (speedup 1.0000x reference)
import functools

import jax
import jax.numpy as jnp
from jax.experimental import pallas as pl
from jax.experimental.pallas import tpu as pltpu

BUMP_STEPS = 32
NUM_FEATURES = 128
NUM_THETAS = 128
N = 10000
B = 128
OUT_W = BUMP_STEPS * NUM_THETAS


def _ect_kernel(lin_ref, x_ref, batch_ref, v_ref, out_ref):
    z = 100.0 * jnp.dot(x_ref[...], v_ref[...],
                        preferred_element_type=jnp.float32)

    batch = batch_ref[0, :]
    gid = jax.lax.broadcasted_iota(jnp.int32, (N, B), 1)
    onehot = batch[:, None] == gid
    oh = jnp.where(onehot, 0.5, 0.0).astype(jnp.float8_e4m3fn)
    half_cnt = 0.5 * jnp.sum(onehot.astype(jnp.float32), axis=0)[:, None]

    for b in range(BUMP_STEPS):
        a_b = 100.0 * lin_ref[b]
        t = jnp.tanh(a_b - z).astype(jnp.float8_e4m3fn)
        part = jax.lax.dot_general(
            oh, t, (((0,), (0,)), ((), ())),
            preferred_element_type=jnp.float32)
        out_ref[:, pl.ds(b * NUM_THETAS, NUM_THETAS)] = part + half_cnt


@functools.partial(jax.jit, static_argnames=())
def kernel(x, batch, v, lin):
    batch2 = batch.astype(jnp.int32).reshape(1, N)
    lin1 = lin.reshape(BUMP_STEPS)
    out = pl.pallas_call(
        _ect_kernel,
        in_specs=[
            pl.BlockSpec(memory_space=pltpu.SMEM),
            pl.BlockSpec((N, NUM_FEATURES), lambda: (0, 0)),
            pl.BlockSpec((1, N), lambda: (0, 0)),
            pl.BlockSpec((NUM_FEATURES, NUM_THETAS), lambda: (0, 0)),
        ],
        out_specs=pl.BlockSpec((B, OUT_W), lambda: (0, 0)),
        out_shape=jax.ShapeDtypeStruct((B, OUT_W), jnp.float32),
    )(lin1, x, batch2, v)
    return out.reshape(B, BUMP_STEPS, NUM_THETAS)
